# Initial kernel scaffold; baseline (speedup 1.0000x reference)
import jax
import jax.numpy as jnp
from jax.experimental import pallas as pl


def kernel(x_user, x_item, edge_attr_r2e, edge_index_r2e, edge_index_follows, W1_0, b1_0, W2_0, b2_0, eps_r2e_0, eps_fol_0, lnw_u_0, lnb_u_0, lnw_i_0, lnb_i_0, W1_1, b1_1, W2_1, b2_1, eps_r2e_1, eps_fol_1, lnw_u_1, lnb_u_1, lnw_i_1, lnb_i_1):
    raise NotImplementedError("write your pallas kernel here")


# speedup vs baseline: 2.7219x; 2.7219x over previous
import functools

import jax
import jax.numpy as jnp
from jax import lax
from jax.experimental import pallas as pl
from jax.experimental.pallas import tpu as pltpu
from jax.experimental.pallas import tpu_sc as plsc

_C = 128
_LANES = 16
_NSUB = 16
_K = 80


def _sc_conv(xu, xi, attr, sr, dr, sf, df, z):
    n_u, c = xu.shape
    n_i, _ = xi.shape
    e = sr.shape[0]
    ept = e // _NSUB
    nchunk = ept // _K
    sw = 400
    assert ept % _K == 0 and n_i % sw == 0 and n_u % sw == 0 and n_i == n_u
    nstripe = n_i // sw

    mesh = plsc.VectorSubcoreMesh(core_axis_name="c", subcore_axis_name="s")

    @functools.partial(
        pl.kernel,
        out_type=(jax.ShapeDtypeStruct((n_i, c), jnp.float32),
                  jax.ShapeDtypeStruct((n_u, c), jnp.float32)),
        mesh=mesh,
        scratch_types=[
            pltpu.VMEM_SHARED((n_i, c), jnp.float32),
            pltpu.VMEM((_K,), jnp.int32),
            pltpu.VMEM((_K,), jnp.int32),
            pltpu.VMEM((_K, c), jnp.float32),
            pltpu.VMEM((_K, c), jnp.float32),
            pltpu.SemaphoreType.DMA,
        ],
    )
    def k(xu_h, xi_h, attr_h, sr_h, dr_h, sf_h, df_h, z_h,
          agg_i_h, agg_u_h, acc, idx_s, idx_d, rows, attrb, sem):
        cid = lax.axis_index("c")
        sid = lax.axis_index("s")
        base = sid * ept

        for j in range((nstripe + _NSUB - 1) // _NSUB):
            st = j * _NSUB + sid

            @pl.when(st < nstripe)
            def _():
                off = pl.multiple_of(st * sw, 8)
                pltpu.sync_copy(z_h.at[pl.ds(off, sw)],
                                acc.at[pl.ds(off, sw)])
        plsc.subcore_barrier()

        @pl.when(cid == 0)
        def _():
            def chunk(t, carry):
                b = pl.multiple_of(base + t * _K, 8)
                pltpu.sync_copy(sr_h.at[pl.ds(b, _K)], idx_s)
                pltpu.sync_copy(dr_h.at[pl.ds(b, _K)], idx_d)
                pltpu.async_copy(xu_h.at[idx_s], rows, sem).wait()
                pltpu.sync_copy(attr_h.at[pl.ds(b, _K)], attrb)

                def ewrow(r, carry2):
                    for cc in range(c // _LANES):
                        sl = pl.ds(cc * _LANES, _LANES)
                        v = rows[r, sl] + attrb[r, sl]
                        rows[r, sl] = jnp.maximum(v, 0.0)
                    return carry2
                lax.fori_loop(0, _K, ewrow, 0, unroll=False)

                pltpu.sync_copy(rows, acc.at[idx_d], add=True)
                return carry
            lax.fori_loop(0, nchunk, chunk, 0, unroll=False)

        @pl.when(cid == 1)
        def _():
            def chunk(t, carry):
                b = pl.multiple_of(base + t * _K, 8)
                pltpu.sync_copy(sf_h.at[pl.ds(b, _K)], idx_s)
                pltpu.sync_copy(df_h.at[pl.ds(b, _K)], idx_d)
                pltpu.async_copy(xi_h.at[idx_s], rows, sem).wait()
                pltpu.sync_copy(rows, acc.at[idx_d], add=True)
                return carry
            lax.fori_loop(0, nchunk, chunk, 0, unroll=False)

        plsc.subcore_barrier()

        for j in range((nstripe + _NSUB - 1) // _NSUB):
            st = j * _NSUB + sid

            @pl.when(st < nstripe)
            def _():
                off = pl.multiple_of(st * sw, 8)

                @pl.when(cid == 0)
                def _():
                    pltpu.sync_copy(acc.at[pl.ds(off, sw)],
                                    agg_i_h.at[pl.ds(off, sw)])

                @pl.when(cid == 1)
                def _():
                    pltpu.sync_copy(acc.at[pl.ds(off, sw)],
                                    agg_u_h.at[pl.ds(off, sw)])

    return k(xu, xi, attr, sr, dr, sf, df, z)


def _mlp_ln(x, agg, eps, w1, b1, w2, b2, lw, lb):
    n, c = x.shape
    h = w1.shape[1]
    blk = 2000
    assert n % blk == 0

    def body(eps_r, x_r, a_r, w1_r, b1_r, w2_r, b2_r, lw_r, lb_r, o_r):
        xx = (1.0 + eps_r[0, 0]) * x_r[...] + a_r[...]
        h1 = jnp.dot(xx, w1_r[...], preferred_element_type=jnp.float32)
        h1 = jnp.maximum(h1 + b1_r[...], 0.0)
        h2 = jnp.dot(h1, w2_r[...], preferred_element_type=jnp.float32)
        h2 = h2 + b2_r[...]
        mu = jnp.mean(h2, axis=-1, keepdims=True)
        d = h2 - mu
        var = jnp.mean(d * d, axis=-1, keepdims=True)
        y = d * lax.rsqrt(var + 1e-5) * lw_r[...] + lb_r[...]
        o_r[...] = jnp.maximum(y, 0.0)

    full = lambda s0, s1: pl.BlockSpec((s0, s1), lambda i: (0, 0))
    return pl.pallas_call(
        body,
        grid=(n // blk,),
        in_specs=[
            full(1, 1),
            pl.BlockSpec((blk, c), lambda i: (i, 0)),
            pl.BlockSpec((blk, c), lambda i: (i, 0)),
            full(c, h), full(1, h), full(h, c), full(1, c),
            full(1, c), full(1, c),
        ],
        out_specs=pl.BlockSpec((blk, c), lambda i: (i, 0)),
        out_shape=jax.ShapeDtypeStruct((n, c), jnp.float32),
    )(eps.reshape(1, 1), x, agg, w1, b1.reshape(1, h), w2,
      b2.reshape(1, c), lw.reshape(1, c), lb.reshape(1, c))


def kernel(x_user, x_item, edge_attr_r2e, edge_index_r2e, edge_index_follows,
           W1_0, b1_0, W2_0, b2_0, eps_r2e_0, eps_fol_0, lnw_u_0, lnb_u_0,
           lnw_i_0, lnb_i_0, W1_1, b1_1, W2_1, b2_1, eps_r2e_1, eps_fol_1,
           lnw_u_1, lnb_u_1, lnw_i_1, lnb_i_1):
    sr, dr = edge_index_r2e[0], edge_index_r2e[1]
    sf, df = edge_index_follows[0], edge_index_follows[1]
    z = jnp.zeros(x_item.shape, jnp.float32)
    params = [
        (W1_0, b1_0, W2_0, b2_0, eps_r2e_0, eps_fol_0,
         lnw_u_0, lnb_u_0, lnw_i_0, lnb_i_0),
        (W1_1, b1_1, W2_1, b2_1, eps_r2e_1, eps_fol_1,
         lnw_u_1, lnb_u_1, lnw_i_1, lnb_i_1),
    ]
    xu, xi = x_user, x_item
    for (w1, b1, w2, b2, eps_r, eps_f, lw_u, lb_u, lw_i, lb_i) in params:
        agg_i, agg_u = _sc_conv(xu, xi, edge_attr_r2e, sr, dr, sf, df, z)
        out_u = _mlp_ln(xu, agg_u, eps_f, w1, b1, w2, b2, lw_u, lb_u)
        out_i = _mlp_ln(xi, agg_i, eps_r, w1, b1, w2, b2, lw_i, lb_i)
        xu, xi = out_u, out_i
    return (xu, xi)


# speedup vs baseline: 3.2651x; 1.1996x over previous
import functools

import jax
import jax.numpy as jnp
from jax import lax
from jax.experimental import pallas as pl
from jax.experimental.pallas import tpu as pltpu
from jax.experimental.pallas import tpu_sc as plsc

_C = 128
_LANES = 16
_NSUB = 16
_K = 40


_RING = 4


def _sc_conv(xu, xi, attr, sr, dr, sf, df, z):
    n_u, c = xu.shape
    n_i, _ = xi.shape
    e = sr.shape[0]
    ept = e // _NSUB
    nchunk = ept // _K
    sw = 400
    assert ept % _K == 0 and n_i % sw == 0 and n_u % sw == 0 and n_i == n_u
    assert nchunk > 2 * _RING
    ngroup = (nchunk - 2) // _RING
    ntail = nchunk - ngroup * _RING
    nstripe = n_i // sw

    mesh = plsc.VectorSubcoreMesh(core_axis_name="c", subcore_axis_name="s")

    @functools.partial(
        pl.kernel,
        out_type=(jax.ShapeDtypeStruct((n_i, c), jnp.float32),
                  jax.ShapeDtypeStruct((n_u, c), jnp.float32)),
        mesh=mesh,
        scratch_types=[
            pltpu.VMEM_SHARED((n_i, c), jnp.float32),
            [pltpu.VMEM((_K, c), jnp.float32) for _ in range(_RING)],
            [pltpu.VMEM((_K, c), jnp.float32) for _ in range(_RING)],
            [pltpu.VMEM((_K,), jnp.int32) for _ in range(_RING)],
            [pltpu.VMEM((_K,), jnp.int32) for _ in range(_RING)],
            pltpu.SemaphoreType.DMA((_RING,)),
            pltpu.SemaphoreType.DMA((_RING,)),
            pltpu.SemaphoreType.DMA((_RING,)),
            pltpu.SemaphoreType.DMA((_RING,)),
        ],
    )
    def k(xu_h, xi_h, attr_h, sr_h, dr_h, sf_h, df_h, z_h,
          agg_i_h, agg_u_h, acc, rows, attrb, isb, idb,
          isem, gsem, asem, ssem):
        cid = lax.axis_index("c")
        sid = lax.axis_index("s")
        base = sid * ept

        for j in range((nstripe + _NSUB - 1) // _NSUB):
            st = j * _NSUB + sid

            @pl.when(st < nstripe)
            def _():
                off = pl.multiple_of(st * sw, 8)
                pltpu.sync_copy(z_h.at[pl.ds(off, sw)],
                                acc.at[pl.ds(off, sw)])
        plsc.subcore_barrier()

        def pipeline(src_h, dst_h, table_h, with_attr):
            def issue_idx(t, b):
                off = pl.multiple_of(base + t * _K, 8)
                pltpu.async_copy(src_h.at[pl.ds(off, _K)], isb[b],
                                 isem.at[b])
                pltpu.async_copy(dst_h.at[pl.ds(off, _K)], idb[b],
                                 isem.at[b])

            def wait_idx(b):
                pltpu.make_async_copy(src_h.at[pl.ds(0, _K)], isb[b],
                                      isem.at[b]).wait()
                pltpu.make_async_copy(dst_h.at[pl.ds(0, _K)], idb[b],
                                      isem.at[b]).wait()

            def issue_loads(t, b):
                pltpu.async_copy(table_h.at[isb[b]], rows[b], gsem.at[b])
                if with_attr:
                    ao = pl.multiple_of(base + t * _K, 8)
                    pltpu.async_copy(attr_h.at[pl.ds(ao, _K)],
                                     attrb[b], asem.at[b])

            def wait_loads(b):
                pltpu.make_async_copy(table_h.at[isb[b]], rows[b],
                                      gsem.at[b]).wait()
                if with_attr:
                    pltpu.make_async_copy(attr_h.at[pl.ds(0, _K)],
                                          attrb[b], asem.at[b]).wait()

            def wait_scatter(b):
                pltpu.make_async_copy(rows[b], acc.at[idb[b]],
                                      ssem.at[b]).wait()

            def process(t, b):
                wait_loads(b)
                if with_attr:
                    def ewrow(r, carry2):
                        for cc in range(c // _LANES):
                            sl = pl.ds(cc * _LANES, _LANES)
                            v = rows[b][r, sl] + attrb[b][r, sl]
                            rows[b][r, sl] = jnp.maximum(v, 0.0)
                        return carry2
                    lax.fori_loop(0, _K, ewrow, 0, unroll=2)
                pltpu.async_copy(rows[b], acc.at[idb[b]], ssem.at[b],
                                 add=True)

            issue_idx(0, 0)
            wait_idx(0)
            issue_loads(0, 0)
            issue_idx(1, 1)

            def group(g, carry):
                for b in range(_RING):
                    t = g * _RING + b
                    s2 = (b + 2) % _RING
                    t2 = t + 2

                    @pl.when(t2 < nchunk)
                    def _():
                        @pl.when(t2 >= _RING)
                        def _():
                            wait_scatter(s2)
                        issue_idx(t2, s2)

                    s1 = (b + 1) % _RING
                    t1 = t + 1

                    @pl.when(t1 < nchunk)
                    def _():
                        wait_idx(s1)
                        issue_loads(t1, s1)

                    process(t, b)
                return carry
            lax.fori_loop(0, ngroup, group, 0, unroll=False)

            for j in range(ntail):
                t = ngroup * _RING + j
                b = t % _RING
                t1 = t + 1
                if t1 < nchunk:
                    wait_idx(t1 % _RING)
                    issue_loads(t1, t1 % _RING)
                process(t, b)
            for b in range(_RING):
                wait_scatter(b)

        @pl.when(cid == 0)
        def _():
            pipeline(sr_h, dr_h, xu_h, True)

        @pl.when(cid == 1)
        def _():
            pipeline(sf_h, df_h, xi_h, False)

        plsc.subcore_barrier()

        for j in range((nstripe + _NSUB - 1) // _NSUB):
            st = j * _NSUB + sid

            @pl.when(st < nstripe)
            def _():
                off = pl.multiple_of(st * sw, 8)

                @pl.when(cid == 0)
                def _():
                    pltpu.sync_copy(acc.at[pl.ds(off, sw)],
                                    agg_i_h.at[pl.ds(off, sw)])

                @pl.when(cid == 1)
                def _():
                    pltpu.sync_copy(acc.at[pl.ds(off, sw)],
                                    agg_u_h.at[pl.ds(off, sw)])

    return k(xu, xi, attr, sr, dr, sf, df, z)


def _mlp_ln(x, agg, eps, w1, b1, w2, b2, lw, lb):
    n, c = x.shape
    h = w1.shape[1]
    blk = 2000
    assert n % blk == 0

    def body(eps_r, x_r, a_r, w1_r, b1_r, w2_r, b2_r, lw_r, lb_r, o_r):
        xx = (1.0 + eps_r[0, 0]) * x_r[...] + a_r[...]
        h1 = jnp.dot(xx, w1_r[...], preferred_element_type=jnp.float32)
        h1 = jnp.maximum(h1 + b1_r[...], 0.0)
        h2 = jnp.dot(h1, w2_r[...], preferred_element_type=jnp.float32)
        h2 = h2 + b2_r[...]
        mu = jnp.mean(h2, axis=-1, keepdims=True)
        d = h2 - mu
        var = jnp.mean(d * d, axis=-1, keepdims=True)
        y = d * lax.rsqrt(var + 1e-5) * lw_r[...] + lb_r[...]
        o_r[...] = jnp.maximum(y, 0.0)

    full = lambda s0, s1: pl.BlockSpec((s0, s1), lambda i: (0, 0))
    return pl.pallas_call(
        body,
        grid=(n // blk,),
        in_specs=[
            full(1, 1),
            pl.BlockSpec((blk, c), lambda i: (i, 0)),
            pl.BlockSpec((blk, c), lambda i: (i, 0)),
            full(c, h), full(1, h), full(h, c), full(1, c),
            full(1, c), full(1, c),
        ],
        out_specs=pl.BlockSpec((blk, c), lambda i: (i, 0)),
        out_shape=jax.ShapeDtypeStruct((n, c), jnp.float32),
    )(eps.reshape(1, 1), x, agg, w1, b1.reshape(1, h), w2,
      b2.reshape(1, c), lw.reshape(1, c), lb.reshape(1, c))


def kernel(x_user, x_item, edge_attr_r2e, edge_index_r2e, edge_index_follows,
           W1_0, b1_0, W2_0, b2_0, eps_r2e_0, eps_fol_0, lnw_u_0, lnb_u_0,
           lnw_i_0, lnb_i_0, W1_1, b1_1, W2_1, b2_1, eps_r2e_1, eps_fol_1,
           lnw_u_1, lnb_u_1, lnw_i_1, lnb_i_1):
    sr, dr = edge_index_r2e[0], edge_index_r2e[1]
    sf, df = edge_index_follows[0], edge_index_follows[1]
    z = jnp.zeros(x_item.shape, jnp.float32)
    params = [
        (W1_0, b1_0, W2_0, b2_0, eps_r2e_0, eps_fol_0,
         lnw_u_0, lnb_u_0, lnw_i_0, lnb_i_0),
        (W1_1, b1_1, W2_1, b2_1, eps_r2e_1, eps_fol_1,
         lnw_u_1, lnb_u_1, lnw_i_1, lnb_i_1),
    ]
    xu, xi = x_user, x_item
    for (w1, b1, w2, b2, eps_r, eps_f, lw_u, lb_u, lw_i, lb_i) in params:
        agg_i, agg_u = _sc_conv(xu, xi, edge_attr_r2e, sr, dr, sf, df, z)
        out_u = _mlp_ln(xu, agg_u, eps_f, w1, b1, w2, b2, lw_u, lb_u)
        out_i = _mlp_ln(xi, agg_i, eps_r, w1, b1, w2, b2, lw_i, lb_i)
        xu, xi = out_u, out_i
    return (xu, xi)
